# Initial kernel scaffold; baseline (speedup 1.0000x reference)
import jax
import jax.numpy as jnp
from jax.experimental import pallas as pl


def kernel(user, pos, neg, pos_beta, neg_beta, weights, neighbor, embs):
    raise NotImplementedError("write your pallas kernel here")


# speedup vs baseline: 2.9489x; 2.9489x over previous
import functools

import jax
import jax.numpy as jnp
from jax import lax
from jax.experimental import pallas as pl
from jax.experimental.pallas import tpu as pltpu, tpu_sc as plsc

USER_N = 100000
ITEM_N = 1000000
HIDDEN = 64
BATCH = 16384
K = 20

_INFO = plsc.get_sparse_core_info()
NC, NS = _INFO.num_cores, _INFO.num_subcores
NW = NC * NS
CHUNK = 128
PER_W = BATCH // NW
N_SUB = PER_W // CHUNK
FLAT = CHUNK * K
NROW = FLAT // CHUNK


def _sc_gather_dot(user, pos, neg, wflat, nbflat, embs):
    mesh = plsc.VectorSubcoreMesh(core_axis_name="c", subcore_axis_name="s")

    @functools.partial(
        pl.kernel,
        out_type=[
            jax.ShapeDtypeStruct((BATCH,), jnp.float32),
            jax.ShapeDtypeStruct((BATCH,), jnp.float32),
            jax.ShapeDtypeStruct((BATCH * K,), jnp.float32),
            jax.ShapeDtypeStruct((BATCH * K,), jnp.float32),
        ],
        mesh=mesh,
        compiler_params=pltpu.CompilerParams(needs_layout_passes=False,
                                             use_tc_tiling_on_sc=False),
        scratch_types=[
            pltpu.VMEM((CHUNK,), jnp.int32),
            pltpu.VMEM((CHUNK,), jnp.int32),
            pltpu.VMEM((CHUNK,), jnp.int32),
            pltpu.VMEM((CHUNK,), jnp.int32),
            pltpu.VMEM((CHUNK, HIDDEN), jnp.float32),
            pltpu.VMEM((CHUNK, HIDDEN), jnp.float32),
            pltpu.VMEM((CHUNK, HIDDEN), jnp.float32),
            pltpu.VMEM((NROW, CHUNK), jnp.int32),
            pltpu.VMEM((NROW, CHUNK), jnp.int32),
            pltpu.VMEM((NROW, CHUNK), jnp.float32),
            pltpu.VMEM((CHUNK, HIDDEN), jnp.float32),
            pltpu.VMEM((CHUNK,), jnp.float32),
            pltpu.VMEM((CHUNK,), jnp.float32),
            pltpu.VMEM((FLAT,), jnp.float32),
            pltpu.SemaphoreType.DMA,
            pltpu.SemaphoreType.DMA,
        ],
    )
    def body(user_r, pos_r, neg_r, wflat_r, nbflat_r, embs_r,
             pos_out, neg_out, nb_out, w_out,
             uidx, pidx, nidx, item, u_rows, p_rows, n_rows,
             eidx2d, flat2d, w_l, nbemb, pos_l, neg_l, nb_l, semA, semB):
        wid = lax.axis_index("s") * NC + lax.axis_index("c")
        lane = lax.iota(jnp.int32, 16)

        def sub_chunk(s, _):
            base = wid * PER_W + s * CHUNK
            pltpu.sync_copy(user_r.at[pl.ds(base, CHUNK)], uidx)
            pltpu.sync_copy(pos_r.at[pl.ds(base, CHUNK)], pidx)
            pltpu.sync_copy(neg_r.at[pl.ds(base, CHUNK)], nidx)

            for v in range(CHUNK // 16):
                pv = pidx[pl.ds(v * 16, 16)]
                it = pv - USER_N
                it = jnp.where(it < 0, it + ITEM_N, it)
                item[pl.ds(v * 16, 16)] = it

            h1 = pltpu.async_copy(embs_r.at[uidx], u_rows, semA)
            h2 = pltpu.async_copy(embs_r.at[pidx], p_rows, semA)
            h3 = pltpu.async_copy(embs_r.at[nidx], n_rows, semA)

            def build_row(j, _):
                for c in range(CHUNK // 16):
                    q = j * CHUNK + c * 16 + lane
                    i_vec = (q * 3277) >> 16
                    k_vec = q - i_vec * K
                    iv = plsc.load_gather(item, [i_vec])
                    f = iv * K + k_vec
                    f = jnp.minimum(jnp.maximum(f, 0), ITEM_N * K - 1)
                    eidx2d[j, pl.ds(c * 16, 16)] = f
                return 0
            lax.fori_loop(0, NROW, build_row, 0)

            def val_row(j, _):
                h4 = pltpu.async_copy(nbflat_r.at[eidx2d.at[j]], flat2d.at[j], semB)
                h5 = pltpu.async_copy(wflat_r.at[eidx2d.at[j]], w_l.at[j], semB)
                h4.wait(); h5.wait()
                return 0
            lax.fori_loop(0, NROW, val_row, 0)

            h1.wait(); h2.wait(); h3.wait()

            def pn_dot(g, _):
                resp = jnp.zeros((16,), jnp.float32)
                resn = jnp.zeros((16,), jnp.float32)
                for u in range(16):
                    e = g * 16 + u
                    accp = jnp.zeros((16,), jnp.float32)
                    accn = jnp.zeros((16,), jnp.float32)
                    for c in range(HIDDEN // 16):
                        uc = u_rows[e, pl.ds(c * 16, 16)]
                        accp = accp + uc * p_rows[e, pl.ds(c * 16, 16)]
                        accn = accn + uc * n_rows[e, pl.ds(c * 16, 16)]
                    resp = jnp.where(lane == u, jnp.sum(accp), resp)
                    resn = jnp.where(lane == u, jnp.sum(accn), resn)
                pos_l[pl.ds(g * 16, 16)] = resp
                neg_l[pl.ds(g * 16, 16)] = resn
                return 0
            lax.fori_loop(0, CHUNK // 16, pn_dot, 0)

            pltpu.sync_copy(pos_l, pos_out.at[pl.ds(base, CHUNK)])
            pltpu.sync_copy(neg_l, neg_out.at[pl.ds(base, CHUNK)])

            def w_row_out(j, _):
                pltpu.sync_copy(w_l.at[j],
                                w_out.at[pl.ds(base * K + j * CHUNK, CHUNK)])
                return 0
            lax.fori_loop(0, NROW, w_row_out, 0)

            def nb_chunk(j, _):
                for c in range(CHUNK // 16):
                    nv = flat2d[j, pl.ds(c * 16, 16)]
                    nv = jnp.minimum(jnp.maximum(nv, 0), USER_N + ITEM_N - 1)
                    flat2d[j, pl.ds(c * 16, 16)] = nv
                pltpu.async_copy(embs_r.at[flat2d.at[j]], nbemb, semB).wait()

                def nb_dot(g, _):
                    res = jnp.zeros((16,), jnp.float32)
                    q = j * CHUNK + g * 16 + lane
                    ev = (q * 3277) >> 16
                    for u in range(16):
                        t = g * 16 + u
                        e = ev[u]
                        acc = jnp.zeros((16,), jnp.float32)
                        for c in range(HIDDEN // 16):
                            acc = acc + (u_rows[e, pl.ds(c * 16, 16)]
                                         * nbemb[t, pl.ds(c * 16, 16)])
                        res = jnp.where(lane == u, jnp.sum(acc), res)
                    nb_l[pl.ds(j * CHUNK + g * 16, 16)] = res
                    return 0
                lax.fori_loop(0, CHUNK // 16, nb_dot, 0)
                return 0
            lax.fori_loop(0, NROW, nb_chunk, 0)

            pltpu.sync_copy(nb_l, nb_out.at[pl.ds(base * K, FLAT)])
            return 0

        lax.fori_loop(0, N_SUB, sub_chunk, 0)

    return body(user, pos, neg, wflat, nbflat, embs)


def _tc_loss_body(p_ref, n_ref, pb_ref, nb_ref, nbl_ref, w_ref, o_ref):
    def sp(y):
        return jnp.maximum(y, 0.0) + jnp.log1p(jnp.exp(-jnp.abs(y)))

    x = p_ref[...]
    y = n_ref[...]
    a = (1.0 + pb_ref[...]) * sp(-x) + (1.0 + nb_ref[...]) * sp(y)
    li = w_ref[...] * sp(-nbl_ref[...])
    o_ref[0, 0] = jnp.sum(a) + 2.5 * jnp.sum(li)


def _tc_loss(pos_l, neg_l, pos_beta, neg_beta, nb_l, w):
    out = pl.pallas_call(
        _tc_loss_body,
        out_shape=jax.ShapeDtypeStruct((1, 1), jnp.float32),
        out_specs=pl.BlockSpec(memory_space=pltpu.SMEM),
    )(
        pos_l.reshape(128, 128), neg_l.reshape(128, 128),
        pos_beta.reshape(128, 128), neg_beta.reshape(128, 128),
        nb_l.reshape(BATCH * K // 128, 128), w.reshape(BATCH * K // 128, 128),
    )
    return jnp.reshape(out, ())


def kernel(user, pos, neg, pos_beta, neg_beta, weights, neighbor, embs):
    pos_l, neg_l, nb_l, w_g = _sc_gather_dot(
        user.astype(jnp.int32), pos.astype(jnp.int32), neg.astype(jnp.int32),
        weights.reshape(-1), neighbor.reshape(-1), embs)
    return _tc_loss(pos_l, neg_l, pos_beta, neg_beta, nb_l, w_g)


# speedup vs baseline: 3.1299x; 1.0614x over previous
import functools

import jax
import jax.numpy as jnp
from jax import lax
from jax.experimental import pallas as pl
from jax.experimental.pallas import tpu as pltpu, tpu_sc as plsc

USER_N = 100000
ITEM_N = 1000000
HIDDEN = 64
BATCH = 16384
K = 20

_INFO = plsc.get_sparse_core_info()
NC, NS = _INFO.num_cores, _INFO.num_subcores
NW = NC * NS
CHUNK = 128
PER_W = BATCH // NW
N_SUB = PER_W // CHUNK
FLAT = CHUNK * K
NROW = FLAT // CHUNK


def _sc_gather_dot(user, pos, neg, wflat, nbflat, embs):
    mesh = plsc.VectorSubcoreMesh(core_axis_name="c", subcore_axis_name="s")

    @functools.partial(
        pl.kernel,
        out_type=[
            jax.ShapeDtypeStruct((BATCH,), jnp.float32),
            jax.ShapeDtypeStruct((BATCH,), jnp.float32),
            jax.ShapeDtypeStruct((BATCH * K,), jnp.float32),
            jax.ShapeDtypeStruct((BATCH * K,), jnp.float32),
        ],
        mesh=mesh,
        compiler_params=pltpu.CompilerParams(needs_layout_passes=False,
                                             use_tc_tiling_on_sc=False),
        scratch_types=[
            pltpu.VMEM((CHUNK,), jnp.int32),
            pltpu.VMEM((CHUNK,), jnp.int32),
            pltpu.VMEM((CHUNK,), jnp.int32),
            pltpu.VMEM((CHUNK,), jnp.int32),
            pltpu.VMEM((CHUNK, HIDDEN), jnp.float32),
            pltpu.VMEM((CHUNK, HIDDEN), jnp.float32),
            pltpu.VMEM((CHUNK, HIDDEN), jnp.float32),
            pltpu.VMEM((NROW, CHUNK), jnp.int32),
            pltpu.VMEM((NROW, CHUNK), jnp.int32),
            pltpu.VMEM((NROW, CHUNK), jnp.float32),
            pltpu.VMEM((CHUNK, HIDDEN), jnp.float32),
            pltpu.VMEM((CHUNK, HIDDEN), jnp.float32),
            pltpu.VMEM((CHUNK,), jnp.float32),
            pltpu.VMEM((CHUNK,), jnp.float32),
            pltpu.VMEM((FLAT,), jnp.float32),
            pltpu.SemaphoreType.DMA,
            pltpu.SemaphoreType.DMA,
            pltpu.SemaphoreType.DMA,
            pltpu.SemaphoreType.DMA,
        ],
    )
    def body(user_r, pos_r, neg_r, wflat_r, nbflat_r, embs_r,
             pos_out, neg_out, nb_out, w_out,
             uidx, pidx, nidx, item, u_rows, p_rows, n_rows,
             eidx2d, flat2d, w_l, nbembA, nbembB, pos_l, neg_l, nb_l,
             semA, semB, semC, semD):
        wid = lax.axis_index("s") * NC + lax.axis_index("c")
        lane = lax.iota(jnp.int32, 16)

        def sub_chunk(s, _):
            base = wid * PER_W + s * CHUNK
            pltpu.sync_copy(user_r.at[pl.ds(base, CHUNK)], uidx)
            pltpu.sync_copy(pos_r.at[pl.ds(base, CHUNK)], pidx)
            pltpu.sync_copy(neg_r.at[pl.ds(base, CHUNK)], nidx)

            for v in range(CHUNK // 16):
                pv = pidx[pl.ds(v * 16, 16)]
                it = pv - USER_N
                it = jnp.where(it < 0, it + ITEM_N, it)
                item[pl.ds(v * 16, 16)] = it

            h1 = pltpu.async_copy(embs_r.at[uidx], u_rows, semA)
            h2 = pltpu.async_copy(embs_r.at[pidx], p_rows, semA)
            h3 = pltpu.async_copy(embs_r.at[nidx], n_rows, semA)

            def build_row(j, _):
                for c in range(CHUNK // 16):
                    q = j * CHUNK + c * 16 + lane
                    i_vec = (q * 3277) >> 16
                    k_vec = q - i_vec * K
                    iv = plsc.load_gather(item, [i_vec])
                    f = iv * K + k_vec
                    f = jnp.minimum(jnp.maximum(f, 0), ITEM_N * K - 1)
                    eidx2d[j, pl.ds(c * 16, 16)] = f
                return 0
            lax.fori_loop(0, NROW, build_row, 0)

            vh = []
            for j in range(NROW):
                vh.append(pltpu.async_copy(nbflat_r.at[eidx2d.at[j]],
                                           flat2d.at[j], semB))
                vh.append(pltpu.async_copy(wflat_r.at[eidx2d.at[j]],
                                           w_l.at[j], semB))

            h1.wait(); h2.wait(); h3.wait()

            def pn_dot(g, _):
                resp = jnp.zeros((16,), jnp.float32)
                resn = jnp.zeros((16,), jnp.float32)
                for u in range(16):
                    e = g * 16 + u
                    accp = jnp.zeros((16,), jnp.float32)
                    accn = jnp.zeros((16,), jnp.float32)
                    for c in range(HIDDEN // 16):
                        uc = u_rows[e, pl.ds(c * 16, 16)]
                        accp = accp + uc * p_rows[e, pl.ds(c * 16, 16)]
                        accn = accn + uc * n_rows[e, pl.ds(c * 16, 16)]
                    resp = jnp.where(lane == u, jnp.sum(accp), resp)
                    resn = jnp.where(lane == u, jnp.sum(accn), resn)
                pos_l[pl.ds(g * 16, 16)] = resp
                neg_l[pl.ds(g * 16, 16)] = resn
                return 0
            lax.fori_loop(0, CHUNK // 16, pn_dot, 0)

            pltpu.sync_copy(pos_l, pos_out.at[pl.ds(base, CHUNK)])
            pltpu.sync_copy(neg_l, neg_out.at[pl.ds(base, CHUNK)])

            for h in vh:
                h.wait()

            wh = [pltpu.async_copy(
                      w_l.at[j], w_out.at[pl.ds(base * K + j * CHUNK, CHUNK)],
                      semA)
                  for j in range(NROW)]

            def clamp_row(j):
                for c in range(CHUNK // 16):
                    nv = flat2d[j, pl.ds(c * 16, 16)]
                    nv = jnp.minimum(jnp.maximum(nv, 0), USER_N + ITEM_N - 1)
                    flat2d[j, pl.ds(c * 16, 16)] = nv

            def nb_dot(j, buf):
                def grp(g, _):
                    res = jnp.zeros((16,), jnp.float32)
                    q = j * CHUNK + g * 16 + lane
                    ev = (q * 3277) >> 16
                    for u in range(16):
                        t = g * 16 + u
                        e = ev[u]
                        acc = jnp.zeros((16,), jnp.float32)
                        for c in range(HIDDEN // 16):
                            acc = acc + (u_rows[e, pl.ds(c * 16, 16)]
                                         * buf[t, pl.ds(c * 16, 16)])
                        res = jnp.where(lane == u, jnp.sum(acc), res)
                    nb_l[pl.ds(j * CHUNK + g * 16, 16)] = res
                    return 0
                lax.fori_loop(0, CHUNK // 16, grp, 0)

            def drain(buf, sem):
                pltpu.make_async_copy(embs_r.at[pl.ds(0, CHUNK)], buf,
                                      sem).wait()

            clamp_row(0)
            pltpu.async_copy(embs_r.at[flat2d.at[0]], nbembA, semC)

            def nb_step(m, _):
                jA = 2 * m
                jB = jA + 1
                clamp_row(jB)
                pltpu.async_copy(embs_r.at[flat2d.at[jB]], nbembB, semD)
                drain(nbembA, semC)
                nb_dot(jA, nbembA)
                jN = jnp.minimum(jA + 2, NROW - 1)
                clamp_row(jN)
                pltpu.async_copy(embs_r.at[flat2d.at[jN]], nbembA, semC)
                drain(nbembB, semD)
                nb_dot(jB, nbembB)
                return 0
            lax.fori_loop(0, NROW // 2, nb_step, 0)
            drain(nbembA, semC)

            pltpu.sync_copy(nb_l, nb_out.at[pl.ds(base * K, FLAT)])
            for h in wh:
                h.wait()
            return 0

        lax.fori_loop(0, N_SUB, sub_chunk, 0)

    return body(user, pos, neg, wflat, nbflat, embs)


def _tc_loss_body(p_ref, n_ref, pb_ref, nb_ref, nbl_ref, w_ref, o_ref):
    def sp(y):
        return jnp.maximum(y, 0.0) + jnp.log1p(jnp.exp(-jnp.abs(y)))

    x = p_ref[...]
    y = n_ref[...]
    a = (1.0 + pb_ref[...]) * sp(-x) + (1.0 + nb_ref[...]) * sp(y)
    li = w_ref[...] * sp(-nbl_ref[...])
    o_ref[0, 0] = jnp.sum(a) + 2.5 * jnp.sum(li)


def _tc_loss(pos_l, neg_l, pos_beta, neg_beta, nb_l, w):
    out = pl.pallas_call(
        _tc_loss_body,
        out_shape=jax.ShapeDtypeStruct((1, 1), jnp.float32),
        out_specs=pl.BlockSpec(memory_space=pltpu.SMEM),
    )(
        pos_l.reshape(128, 128), neg_l.reshape(128, 128),
        pos_beta.reshape(128, 128), neg_beta.reshape(128, 128),
        nb_l.reshape(BATCH * K // 128, 128), w.reshape(BATCH * K // 128, 128),
    )
    return jnp.reshape(out, ())


def kernel(user, pos, neg, pos_beta, neg_beta, weights, neighbor, embs):
    pos_l, neg_l, nb_l, w_g = _sc_gather_dot(
        user.astype(jnp.int32), pos.astype(jnp.int32), neg.astype(jnp.int32),
        weights.reshape(-1), neighbor.reshape(-1), embs)
    return _tc_loss(pos_l, neg_l, pos_beta, neg_beta, nb_l, w_g)
